# baseline (speedup 1.0000x reference)
import functools

import jax
import jax.numpy as jnp
from jax import lax
from jax.experimental import pallas as pl
from jax.experimental.pallas import tpu as pltpu
from jax.experimental.pallas import tpu_sc as plsc

N = 10000
IN = 128
HID = 32
OUT = 16
H1 = 8
E = 320000

_L = 16
_G = 80
_NSUB = 16


def _iota16():
    return lax.broadcasted_iota(jnp.int32, (_L,), 0)


def _group_compute(A, B, D, wbuf, attv, g, n_heads, chw):
    rows = g * _L + _iota16()

    zero = jnp.zeros((_L,), jnp.float32)
    ws = []
    for h in range(n_heads):
        base = jnp.full((_L,), h * chw, jnp.int32)

        def p1(k, car):
            accs = list(car)
            for cc in range(8):
                colv = base + (k * 8 + cc)
                a = plsc.load_gather(A, [rows, colv])
                b = plsc.load_gather(B, [rows, colv])
                av = plsc.load_gather(attv, [colv])
                t = a + b
                t = jnp.maximum(t, t * 0.2)
                accs[cc % 4] = accs[cc % 4] + av * t
            return tuple(accs)

        accs = lax.fori_loop(0, chw // 8, p1, (zero, zero, zero, zero))
        ws.append(jnp.exp((accs[0] + accs[1]) + (accs[2] + accs[3])))

    for h in range(n_heads):
        wh = ws[h]
        base = jnp.full((_L,), h * chw, jnp.int32)

        def p2(k, _):
            for cc in range(8):
                colv = base + (k * 8 + cc)
                col = plsc.load_gather(A, [rows, colv])
                plsc.store_scatter(A, [rows, colv], col * wh)
            return 0

        lax.fori_loop(0, chw // 8, p2, 0)

    for l in range(_L):
        colv = jnp.full((_L,), l, jnp.int32)
        val = ws[l] if l < n_heads else zero
        plsc.store_scatter(D, [rows, colv], val)


def _sc_core_run(xl, xr, n0, d0, nout, dout, att, src_h, dst_h,
                 numer_sh, denom_sh, A3, B3, D3, dsml3, srcstg, dststg,
                 wbuf, attv, gsa, gsb, ssn, ssd,
                 sid, edge_base, ept, n_heads, chw, G, stg):
    nchunks = ept // G
    base0 = edge_base + sid * ept

    pltpu.sync_copy(att, attv)

    @pl.when(sid == 0)
    def _():
        pltpu.sync_copy(n0, numer_sh)
        pltpu.sync_copy(d0, denom_sh)

    def stage(si):
        pltpu.sync_copy(src_h.at[pl.ds(base0 + si * stg * G, stg * G)],
                        srcstg)
        pltpu.sync_copy(dst_h.at[pl.ds(base0 + si * stg * G, stg * G)],
                        dststg)

    def fill_dsml(i, b):
        off = (i % stg) * G
        for j in range(G // _L):
            dsml3[b][pl.ds(j * _L, _L)] = dststg[pl.ds(off + j * _L, _L)]

    def gather_descs(i, b):
        off = (i % stg) * G
        return (
            pltpu.make_async_copy(xl.at[srcstg.at[pl.ds(off, G)]],
                                  A3[b], gsa[b]),
            pltpu.make_async_copy(xr.at[dststg.at[pl.ds(off, G)]],
                                  B3[b], gsb[b]),
        )

    def issue(i, b):
        fill_dsml(i, b)
        da, db = gather_descs(i, b)
        da.start()
        db.start()

    def wait_gathers(i, b):
        da, db = gather_descs(i, b)
        da.wait()
        db.wait()

    def issue_scatter(b):
        pltpu.async_copy(A3[b], numer_sh.at[dsml3[b]], ssn[b], add=True)
        pltpu.async_copy(D3[b], denom_sh.at[dsml3[b]], ssd[b], add=True)

    def wait_scatter(b):
        pltpu.make_async_copy(A3[b], numer_sh.at[dsml3[b]], ssn[b]).wait()
        pltpu.make_async_copy(D3[b], denom_sh.at[dsml3[b]], ssd[b]).wait()

    plsc.subcore_barrier()
    stage(0)
    issue(0, 0)

    def step(i, b):
        nb = (b + 1) % 3
        wait_gathers(i, b)

        @pl.when(i + 1 < nchunks)
        def _():
            @pl.when(i >= 2)
            def _():
                wait_scatter(nb)

            @pl.when((i + 1) % stg == 0)
            def _():
                stage((i + 1) // stg)

            issue(i + 1, nb)

        def group(g, _):
            _group_compute(A3[b], B3[b], D3[b], wbuf, attv, g, n_heads, chw)
            return 0

        lax.fori_loop(0, G // _L, group, 0)
        issue_scatter(b)

    def triple(p, _):
        step(3 * p, 0)
        step(3 * p + 1, 1)
        step(3 * p + 2, 2)
        return 0

    lax.fori_loop(0, nchunks // 3, triple, 0)
    for k in range(nchunks % 3):
        step(3 * (nchunks // 3) + k, k)

    for k in range(3):
        wait_scatter((nchunks - 3 + k) % 3)
    plsc.subcore_barrier()

    @pl.when(sid == 0)
    def _():
        pltpu.sync_copy(numer_sh, nout)
        pltpu.sync_copy(denom_sh, dout)


_G1 = 32
_G2 = 80
_STG = 25


def _l1_body(xl0, xl1, xr0, xr1, n00, n01, d00, d01, src_h, dst_h, att0, att1,
             nout0, nout1, dout0, dout1,
             numer_sh, denom_sh, A3, B3, D3, dsml3, srcstg, dststg,
             wbuf, attv, gsa, gsb, ssn, ssd):
    cid = lax.axis_index("c")
    sid = lax.axis_index("s")
    ept = E // _NSUB

    @pl.when(cid == 0)
    def _():
        _sc_core_run(xl0, xr0, n00, d00, nout0, dout0, att0, src_h, dst_h,
                     numer_sh, denom_sh, A3, B3, D3, dsml3, srcstg, dststg,
                     wbuf, attv, gsa, gsb, ssn, ssd,
                     sid, 0, ept, 4, HID, _G1, _STG)

    @pl.when(cid == 1)
    def _():
        _sc_core_run(xl1, xr1, n01, d01, nout1, dout1, att1, src_h, dst_h,
                     numer_sh, denom_sh, A3, B3, D3, dsml3, srcstg, dststg,
                     wbuf, attv, gsa, gsb, ssn, ssd,
                     sid, 0, ept, 4, HID, _G1, _STG)


def _l2_body(xl, xr, n0a, n0b, d0a, d0b, src_h, dst_h, att,
             nout0, nout1, dout0, dout1,
             numer_sh, denom_sh, A3, B3, D3, dsml3, srcstg, dststg,
             wbuf, attv, gsa, gsb, ssn, ssd):
    cid = lax.axis_index("c")
    sid = lax.axis_index("s")
    ept = (E // 2) // _NSUB

    @pl.when(cid == 0)
    def _():
        _sc_core_run(xl, xr, n0a, d0a, nout0, dout0, att, src_h, dst_h,
                     numer_sh, denom_sh, A3, B3, D3, dsml3, srcstg, dststg,
                     wbuf, attv, gsa, gsb, ssn, ssd,
                     sid, 0, ept, 1, OUT, _G2, _STG)

    @pl.when(cid == 1)
    def _():
        _sc_core_run(xl, xr, n0b, d0b, nout1, dout1, att, src_h, dst_h,
                     numer_sh, denom_sh, A3, B3, D3, dsml3, srcstg, dststg,
                     wbuf, attv, gsa, gsb, ssn, ssd,
                     sid, E // 2, ept, 1, OUT, _G2, _STG)


def _make_edge_pass(body, ch, G, stg):
    f32 = jnp.float32
    i32 = jnp.int32
    mesh = plsc.VectorSubcoreMesh(core_axis_name="c", subcore_axis_name="s")
    return pl.kernel(
        body,
        out_type=(
            jax.ShapeDtypeStruct((N, ch), f32),
            jax.ShapeDtypeStruct((N, ch), f32),
            jax.ShapeDtypeStruct((N, _L), f32),
            jax.ShapeDtypeStruct((N, _L), f32),
        ),
        mesh=mesh,
        scratch_types=[
            pltpu.VMEM_SHARED((N, ch), f32),
            pltpu.VMEM_SHARED((N, _L), f32),
            [pltpu.VMEM((G, ch), f32)] * 3,
            [pltpu.VMEM((G, ch), f32)] * 3,
            [pltpu.VMEM((G, _L), f32)] * 3,
            [pltpu.VMEM((G,), i32)] * 3,
            pltpu.VMEM((stg * G,), i32),
            pltpu.VMEM((stg * G,), i32),
            pltpu.VMEM((8, _L), f32),
            pltpu.VMEM((ch,), f32),
            [pltpu.SemaphoreType.DMA] * 3,
            [pltpu.SemaphoreType.DMA] * 3,
            [pltpu.SemaphoreType.DMA] * 3,
            [pltpu.SemaphoreType.DMA] * 3,
        ],
        compiler_params=pltpu.CompilerParams(use_tc_tiling_on_sc=False,
                                             needs_layout_passes=False),
    )


_l1_pass = _make_edge_pass(_l1_body, 128, _G1, _STG)
_l2_pass = _make_edge_pass(_l2_body, OUT, _G2, _STG)


def _decoder_body(z_row, z_col, out_ref):
    acc = lax.dot_general(z_row[...], z_col[...], (((1,), (1,)), ((), ())),
                          preferred_element_type=jnp.float32)
    out_ref[...] = jax.nn.sigmoid(acc)


def _decoder(zp):
    BM = 512
    BN = 512
    grid = (pl.cdiv(N, BM), pl.cdiv(N, BN))
    return pl.pallas_call(
        _decoder_body,
        grid=grid,
        in_specs=[
            pl.BlockSpec((BM, 128), lambda i, j: (i, 0)),
            pl.BlockSpec((BN, 128), lambda i, j: (j, 0)),
        ],
        out_specs=pl.BlockSpec((BM, BN), lambda i, j: (i, j)),
        out_shape=jax.ShapeDtypeStruct((N, N), jnp.float32),
    )(zp, zp)


def _self_loop_init(xl, xr, att, n_heads, chw):
    xlh = xl.reshape(N, n_heads, chw)
    xrh = xr.reshape(N, n_heads, chw)
    t = xlh + xrh
    t = jnp.where(t >= 0.0, t, t * 0.2)
    sw = jnp.exp((t * att.reshape(1, n_heads, chw)).sum(-1))
    numer0 = (sw[:, :, None] * xlh).reshape(N, n_heads * chw)
    lane = jnp.arange(_L)
    denom0 = jnp.where(lane[None, :] < n_heads,
                       sw[:, lane % n_heads], 0.0).astype(jnp.float32)
    return numer0, denom0


def kernel(x, edge_index, W1l, b1l, W1r, b1r, att1, bias1,
           W2l, b2l, W2r, b2r, att2, bias2):
    src = edge_index[0]
    dst = edge_index[1]

    xl = x @ W1l + b1l
    xr = x @ W1r + b1r
    att1f = att1.reshape(H1 * HID)

    n00, d00 = _self_loop_init(xl[:, :128], xr[:, :128], att1f[:128], 4, HID)
    n01, d01 = _self_loop_init(xl[:, 128:], xr[:, 128:], att1f[128:], 4, HID)

    nout0, nout1, dout0, dout1 = _l1_pass(
        xl[:, :128], xl[:, 128:], xr[:, :128], xr[:, 128:],
        n00, n01, d00, d01, src, dst, att1f[:128], att1f[128:])

    numer = jnp.concatenate([nout0, nout1], axis=1).reshape(N, H1, HID)
    denom = jnp.concatenate([dout0[:, :4], dout1[:, :4]], axis=1)
    h1 = (numer / (denom[:, :, None] + 1e-16)).reshape(N, H1 * HID) + bias1

    xl2 = h1 @ W2l + b2l
    xr2 = h1 @ W2r + b2r
    att2f = att2.reshape(OUT)

    n0a, d0a = _self_loop_init(xl2, xr2, att2f, 1, OUT)
    zN16 = jnp.zeros((N, OUT), jnp.float32)
    zN = jnp.zeros((N, _L), jnp.float32)

    n20, n21, d20, d21 = _l2_pass(
        xl2, xr2, n0a, zN16, d0a, zN, src, dst, att2f)

    h2 = (n20 + n21) / (d20[:, :1] + d21[:, :1] + 1e-16) + bias2

    z = h2 / jnp.maximum(jnp.linalg.norm(h2, axis=1, keepdims=True), 1e-12)
    zp = jnp.pad(z, ((0, 0), (0, 128 - OUT)))
    A_pred = _decoder(zp)
    return (A_pred, z)


# speedup vs baseline: 1.0121x; 1.0121x over previous
import functools

import jax
import jax.numpy as jnp
from jax import lax
from jax.experimental import pallas as pl
from jax.experimental.pallas import tpu as pltpu
from jax.experimental.pallas import tpu_sc as plsc

N = 10000
IN = 128
HID = 32
OUT = 16
H1 = 8
E = 320000

_L = 16
_G = 80
_NSUB = 16


def _iota16():
    return lax.broadcasted_iota(jnp.int32, (_L,), 0)


def _group_compute(A, B, D, wbuf, attv, g, n_heads, chw):
    rows = g * _L + _iota16()

    zero = jnp.zeros((_L,), jnp.float32)
    ws = []
    for h in range(n_heads):
        base = jnp.full((_L,), h * chw, jnp.int32)

        def p1(k, car):
            cb = car[0]
            accs = list(car[1:])
            for cc in range(8):
                colv = cb + cc
                a = plsc.load_gather(A, [rows, colv])
                b = plsc.load_gather(B, [rows, colv])
                av = plsc.load_gather(attv, [colv])
                t = a + b
                t = jnp.maximum(t, t * 0.2)
                accs[cc % 4] = accs[cc % 4] + av * t
            return (cb + 8, *accs)

        accs = lax.fori_loop(0, chw // 8, p1,
                             (base, zero, zero, zero, zero))[1:]
        ws.append(jnp.exp((accs[0] + accs[1]) + (accs[2] + accs[3])))

    for h in range(n_heads):
        wh = ws[h]
        base = jnp.full((_L,), h * chw, jnp.int32)

        def p2(k, cb):
            for cc in range(8):
                colv = cb + cc
                col = plsc.load_gather(A, [rows, colv])
                plsc.store_scatter(A, [rows, colv], col * wh)
            return cb + 8

        lax.fori_loop(0, chw // 8, p2, base)

    for l in range(_L):
        colv = jnp.full((_L,), l, jnp.int32)
        val = ws[l] if l < n_heads else zero
        plsc.store_scatter(D, [rows, colv], val)


def _sc_core_run(xl, xr, n0, d0, nout, dout, att, src_h, dst_h,
                 numer_sh, denom_sh, A3, B3, D3, dsml3, srcstg, dststg,
                 wbuf, attv, gsa, gsb, ssn, ssd,
                 sid, edge_base, ept, n_heads, chw, G, stg):
    nchunks = ept // G
    base0 = edge_base + sid * ept

    pltpu.sync_copy(att, attv)

    @pl.when(sid == 0)
    def _():
        pltpu.sync_copy(n0, numer_sh)
        pltpu.sync_copy(d0, denom_sh)

    def stage(si):
        pltpu.sync_copy(src_h.at[pl.ds(base0 + si * stg * G, stg * G)],
                        srcstg)
        pltpu.sync_copy(dst_h.at[pl.ds(base0 + si * stg * G, stg * G)],
                        dststg)

    def fill_dsml(i, b):
        off = (i % stg) * G
        for j in range(G // _L):
            dsml3[b][pl.ds(j * _L, _L)] = dststg[pl.ds(off + j * _L, _L)]

    def gather_descs(i, b):
        off = (i % stg) * G
        return (
            pltpu.make_async_copy(xl.at[srcstg.at[pl.ds(off, G)]],
                                  A3[b], gsa[b]),
            pltpu.make_async_copy(xr.at[dststg.at[pl.ds(off, G)]],
                                  B3[b], gsb[b]),
        )

    def issue(i, b):
        fill_dsml(i, b)
        da, db = gather_descs(i, b)
        da.start()
        db.start()

    def wait_gathers(i, b):
        da, db = gather_descs(i, b)
        da.wait()
        db.wait()

    def issue_scatter(b):
        pltpu.async_copy(A3[b], numer_sh.at[dsml3[b]], ssn[b], add=True)
        pltpu.async_copy(D3[b], denom_sh.at[dsml3[b]], ssd[b], add=True)

    def wait_scatter(b):
        pltpu.make_async_copy(A3[b], numer_sh.at[dsml3[b]], ssn[b]).wait()
        pltpu.make_async_copy(D3[b], denom_sh.at[dsml3[b]], ssd[b]).wait()

    plsc.subcore_barrier()
    stage(0)
    issue(0, 0)

    def step(i, b):
        nb = (b + 1) % 3
        wait_gathers(i, b)

        @pl.when(i + 1 < nchunks)
        def _():
            @pl.when(i >= 2)
            def _():
                wait_scatter(nb)

            @pl.when((i + 1) % stg == 0)
            def _():
                stage((i + 1) // stg)

            issue(i + 1, nb)

        def group(g, _):
            _group_compute(A3[b], B3[b], D3[b], wbuf, attv, g, n_heads, chw)
            return 0

        lax.fori_loop(0, G // _L, group, 0)
        issue_scatter(b)

    def triple(p, _):
        step(3 * p, 0)
        step(3 * p + 1, 1)
        step(3 * p + 2, 2)
        return 0

    lax.fori_loop(0, nchunks // 3, triple, 0)
    for k in range(nchunks % 3):
        step(3 * (nchunks // 3) + k, k)

    for k in range(3):
        wait_scatter((nchunks - 3 + k) % 3)
    plsc.subcore_barrier()

    @pl.when(sid == 0)
    def _():
        pltpu.sync_copy(numer_sh, nout)
        pltpu.sync_copy(denom_sh, dout)


_G1 = 32
_G2 = 80
_STG = 25


def _l1_body(xl0, xl1, xr0, xr1, n00, n01, d00, d01, src_h, dst_h, att0, att1,
             nout0, nout1, dout0, dout1,
             numer_sh, denom_sh, A3, B3, D3, dsml3, srcstg, dststg,
             wbuf, attv, gsa, gsb, ssn, ssd):
    cid = lax.axis_index("c")
    sid = lax.axis_index("s")
    ept = E // _NSUB

    @pl.when(cid == 0)
    def _():
        _sc_core_run(xl0, xr0, n00, d00, nout0, dout0, att0, src_h, dst_h,
                     numer_sh, denom_sh, A3, B3, D3, dsml3, srcstg, dststg,
                     wbuf, attv, gsa, gsb, ssn, ssd,
                     sid, 0, ept, 4, HID, _G1, _STG)

    @pl.when(cid == 1)
    def _():
        _sc_core_run(xl1, xr1, n01, d01, nout1, dout1, att1, src_h, dst_h,
                     numer_sh, denom_sh, A3, B3, D3, dsml3, srcstg, dststg,
                     wbuf, attv, gsa, gsb, ssn, ssd,
                     sid, 0, ept, 4, HID, _G1, _STG)


def _l2_body(xl, xr, n0a, n0b, d0a, d0b, src_h, dst_h, att,
             nout0, nout1, dout0, dout1,
             numer_sh, denom_sh, A3, B3, D3, dsml3, srcstg, dststg,
             wbuf, attv, gsa, gsb, ssn, ssd):
    cid = lax.axis_index("c")
    sid = lax.axis_index("s")
    ept = (E // 2) // _NSUB

    @pl.when(cid == 0)
    def _():
        _sc_core_run(xl, xr, n0a, d0a, nout0, dout0, att, src_h, dst_h,
                     numer_sh, denom_sh, A3, B3, D3, dsml3, srcstg, dststg,
                     wbuf, attv, gsa, gsb, ssn, ssd,
                     sid, 0, ept, 1, OUT, _G2, _STG)

    @pl.when(cid == 1)
    def _():
        _sc_core_run(xl, xr, n0b, d0b, nout1, dout1, att, src_h, dst_h,
                     numer_sh, denom_sh, A3, B3, D3, dsml3, srcstg, dststg,
                     wbuf, attv, gsa, gsb, ssn, ssd,
                     sid, E // 2, ept, 1, OUT, _G2, _STG)


def _make_edge_pass(body, ch, G, stg):
    f32 = jnp.float32
    i32 = jnp.int32
    mesh = plsc.VectorSubcoreMesh(core_axis_name="c", subcore_axis_name="s")
    return pl.kernel(
        body,
        out_type=(
            jax.ShapeDtypeStruct((N, ch), f32),
            jax.ShapeDtypeStruct((N, ch), f32),
            jax.ShapeDtypeStruct((N, _L), f32),
            jax.ShapeDtypeStruct((N, _L), f32),
        ),
        mesh=mesh,
        scratch_types=[
            pltpu.VMEM_SHARED((N, ch), f32),
            pltpu.VMEM_SHARED((N, _L), f32),
            [pltpu.VMEM((G, ch), f32)] * 3,
            [pltpu.VMEM((G, ch), f32)] * 3,
            [pltpu.VMEM((G, _L), f32)] * 3,
            [pltpu.VMEM((G,), i32)] * 3,
            pltpu.VMEM((stg * G,), i32),
            pltpu.VMEM((stg * G,), i32),
            pltpu.VMEM((8, _L), f32),
            pltpu.VMEM((ch,), f32),
            [pltpu.SemaphoreType.DMA] * 3,
            [pltpu.SemaphoreType.DMA] * 3,
            [pltpu.SemaphoreType.DMA] * 3,
            [pltpu.SemaphoreType.DMA] * 3,
        ],
        compiler_params=pltpu.CompilerParams(use_tc_tiling_on_sc=False,
                                             needs_layout_passes=False),
    )


_l1_pass = _make_edge_pass(_l1_body, 128, _G1, _STG)
_l2_pass = _make_edge_pass(_l2_body, OUT, _G2, _STG)


def _decoder_body(z_row, z_col, out_ref):
    acc = lax.dot_general(z_row[...], z_col[...], (((1,), (1,)), ((), ())),
                          preferred_element_type=jnp.float32)
    out_ref[...] = jax.nn.sigmoid(acc)


def _decoder(zp):
    BM = 512
    BN = 512
    grid = (pl.cdiv(N, BM), pl.cdiv(N, BN))
    return pl.pallas_call(
        _decoder_body,
        grid=grid,
        in_specs=[
            pl.BlockSpec((BM, 128), lambda i, j: (i, 0)),
            pl.BlockSpec((BN, 128), lambda i, j: (j, 0)),
        ],
        out_specs=pl.BlockSpec((BM, BN), lambda i, j: (i, j)),
        out_shape=jax.ShapeDtypeStruct((N, N), jnp.float32),
    )(zp, zp)


def _self_loop_init(xl, xr, att, n_heads, chw):
    xlh = xl.reshape(N, n_heads, chw)
    xrh = xr.reshape(N, n_heads, chw)
    t = xlh + xrh
    t = jnp.where(t >= 0.0, t, t * 0.2)
    sw = jnp.exp((t * att.reshape(1, n_heads, chw)).sum(-1))
    numer0 = (sw[:, :, None] * xlh).reshape(N, n_heads * chw)
    lane = jnp.arange(_L)
    denom0 = jnp.where(lane[None, :] < n_heads,
                       sw[:, lane % n_heads], 0.0).astype(jnp.float32)
    return numer0, denom0


def kernel(x, edge_index, W1l, b1l, W1r, b1r, att1, bias1,
           W2l, b2l, W2r, b2r, att2, bias2):
    src = edge_index[0]
    dst = edge_index[1]

    xl = x @ W1l + b1l
    xr = x @ W1r + b1r
    att1f = att1.reshape(H1 * HID)

    n00, d00 = _self_loop_init(xl[:, :128], xr[:, :128], att1f[:128], 4, HID)
    n01, d01 = _self_loop_init(xl[:, 128:], xr[:, 128:], att1f[128:], 4, HID)

    nout0, nout1, dout0, dout1 = _l1_pass(
        xl[:, :128], xl[:, 128:], xr[:, :128], xr[:, 128:],
        n00, n01, d00, d01, src, dst, att1f[:128], att1f[128:])

    numer = jnp.concatenate([nout0, nout1], axis=1).reshape(N, H1, HID)
    denom = jnp.concatenate([dout0[:, :4], dout1[:, :4]], axis=1)
    h1 = (numer / (denom[:, :, None] + 1e-16)).reshape(N, H1 * HID) + bias1

    xl2 = h1 @ W2l + b2l
    xr2 = h1 @ W2r + b2r
    att2f = att2.reshape(OUT)

    n0a, d0a = _self_loop_init(xl2, xr2, att2f, 1, OUT)
    zN16 = jnp.zeros((N, OUT), jnp.float32)
    zN = jnp.zeros((N, _L), jnp.float32)

    n20, n21, d20, d21 = _l2_pass(
        xl2, xr2, n0a, zN16, d0a, zN, src, dst, att2f)

    h2 = (n20 + n21) / (d20[:, :1] + d21[:, :1] + 1e-16) + bias2

    z = h2 / jnp.maximum(jnp.linalg.norm(h2, axis=1, keepdims=True), 1e-12)
    zp = jnp.pad(z, ((0, 0), (0, 128 - OUT)))
    A_pred = _decoder(zp)
    return (A_pred, z)


# speedup vs baseline: 2.9976x; 2.9618x over previous
import functools

import jax
import jax.numpy as jnp
from jax import lax
from jax.experimental import pallas as pl
from jax.experimental.pallas import tpu as pltpu
from jax.experimental.pallas import tpu_sc as plsc

N = 10000
IN = 128
HID = 32
OUT = 16
H1 = 8
E = 320000

_L = 16
_G = 80
_NSUB = 16


def _iota16():
    return lax.broadcasted_iota(jnp.int32, (_L,), 0)


def _group_compute(A, B, D, wbuf, attv, g, n_heads, chw):
    rows = g * _L + _iota16()

    lane = _iota16()
    lanemod = lane % n_heads
    headmask = (lane < n_heads).astype(jnp.float32)
    n_cb = (n_heads * chw) // _L
    zero = jnp.zeros((_L,), jnp.float32)

    ws = []
    for h in range(n_heads):
        accs = [zero, zero, zero, zero]
        for c in range(chw):
            rot = (lane + c) & (chw - 1)
            colv = rot + (h * chw)
            a = plsc.load_gather(A, [rows, colv])
            b = plsc.load_gather(B, [rows, colv])
            av = plsc.load_gather(attv, [colv])
            t = a + b
            t = jnp.maximum(t, t * 0.2)
            accs[c % 4] = accs[c % 4] + av * t
        w = jnp.exp((accs[0] + accs[1]) + (accs[2] + accs[3]))
        wbuf[h, :] = w
        ws.append(w)

    for e in range(_L):
        eab = g * _L + e
        ev = jnp.full((_L,), e, jnp.int32)
        for cb in range(n_cb):
            h = (cb * _L) // chw
            sl = pl.ds(cb * _L, _L)
            A[eab, sl] = A[eab, sl] * ws[h][e]
        wrow = plsc.load_gather(wbuf, [lanemod, ev])
        D[eab, :] = wrow * headmask


def _sc_core_run(xl, xr, n0, d0, nout, dout, att, src_h, dst_h,
                 numer_sh, denom_sh, A3, B3, D3, dsml3, srcstg, dststg,
                 wbuf, attv, gsa, gsb, ssn, ssd,
                 sid, edge_base, ept, n_heads, chw, G, stg):
    nchunks = ept // G
    base0 = edge_base + sid * ept

    pltpu.sync_copy(att, attv)

    @pl.when(sid == 0)
    def _():
        pltpu.sync_copy(n0, numer_sh)
        pltpu.sync_copy(d0, denom_sh)

    def stage(si):
        pltpu.sync_copy(src_h.at[pl.ds(base0 + si * stg * G, stg * G)],
                        srcstg)
        pltpu.sync_copy(dst_h.at[pl.ds(base0 + si * stg * G, stg * G)],
                        dststg)

    def fill_dsml(i, b):
        off = (i % stg) * G
        for j in range(G // _L):
            dsml3[b][pl.ds(j * _L, _L)] = dststg[pl.ds(off + j * _L, _L)]

    def gather_descs(i, b):
        off = (i % stg) * G
        return (
            pltpu.make_async_copy(xl.at[srcstg.at[pl.ds(off, G)]],
                                  A3[b], gsa[b]),
            pltpu.make_async_copy(xr.at[dststg.at[pl.ds(off, G)]],
                                  B3[b], gsb[b]),
        )

    def issue(i, b):
        fill_dsml(i, b)
        da, db = gather_descs(i, b)
        da.start()
        db.start()

    def wait_gathers(i, b):
        da, db = gather_descs(i, b)
        da.wait()
        db.wait()

    def issue_scatter(b):
        pltpu.async_copy(A3[b], numer_sh.at[dsml3[b]], ssn[b], add=True)
        pltpu.async_copy(D3[b], denom_sh.at[dsml3[b]], ssd[b], add=True)

    def wait_scatter(b):
        pltpu.make_async_copy(A3[b], numer_sh.at[dsml3[b]], ssn[b]).wait()
        pltpu.make_async_copy(D3[b], denom_sh.at[dsml3[b]], ssd[b]).wait()

    plsc.subcore_barrier()
    stage(0)
    issue(0, 0)

    def step(i, b):
        nb = (b + 1) % 3
        wait_gathers(i, b)

        @pl.when(i + 1 < nchunks)
        def _():
            @pl.when(i >= 2)
            def _():
                wait_scatter(nb)

            @pl.when((i + 1) % stg == 0)
            def _():
                stage((i + 1) // stg)

            issue(i + 1, nb)

        def group(g, _):
            _group_compute(A3[b], B3[b], D3[b], wbuf, attv, g, n_heads, chw)
            return 0

        lax.fori_loop(0, G // _L, group, 0)
        issue_scatter(b)

    def triple(p, _):
        step(3 * p, 0)
        step(3 * p + 1, 1)
        step(3 * p + 2, 2)
        return 0

    lax.fori_loop(0, nchunks // 3, triple, 0)
    for k in range(nchunks % 3):
        step(3 * (nchunks // 3) + k, k)

    for k in range(3):
        wait_scatter((nchunks - 3 + k) % 3)
    plsc.subcore_barrier()

    @pl.when(sid == 0)
    def _():
        pltpu.sync_copy(numer_sh, nout)
        pltpu.sync_copy(denom_sh, dout)


_G1 = 32
_G2 = 80
_STG = 25


def _l1_body(xl0, xl1, xr0, xr1, n00, n01, d00, d01, src_h, dst_h, att0, att1,
             nout0, nout1, dout0, dout1,
             numer_sh, denom_sh, A3, B3, D3, dsml3, srcstg, dststg,
             wbuf, attv, gsa, gsb, ssn, ssd):
    cid = lax.axis_index("c")
    sid = lax.axis_index("s")
    ept = E // _NSUB

    @pl.when(cid == 0)
    def _():
        _sc_core_run(xl0, xr0, n00, d00, nout0, dout0, att0, src_h, dst_h,
                     numer_sh, denom_sh, A3, B3, D3, dsml3, srcstg, dststg,
                     wbuf, attv, gsa, gsb, ssn, ssd,
                     sid, 0, ept, 4, HID, _G1, _STG)

    @pl.when(cid == 1)
    def _():
        _sc_core_run(xl1, xr1, n01, d01, nout1, dout1, att1, src_h, dst_h,
                     numer_sh, denom_sh, A3, B3, D3, dsml3, srcstg, dststg,
                     wbuf, attv, gsa, gsb, ssn, ssd,
                     sid, 0, ept, 4, HID, _G1, _STG)


def _l2_body(xl, xr, n0a, n0b, d0a, d0b, src_h, dst_h, att,
             nout0, nout1, dout0, dout1,
             numer_sh, denom_sh, A3, B3, D3, dsml3, srcstg, dststg,
             wbuf, attv, gsa, gsb, ssn, ssd):
    cid = lax.axis_index("c")
    sid = lax.axis_index("s")
    ept = (E // 2) // _NSUB

    @pl.when(cid == 0)
    def _():
        _sc_core_run(xl, xr, n0a, d0a, nout0, dout0, att, src_h, dst_h,
                     numer_sh, denom_sh, A3, B3, D3, dsml3, srcstg, dststg,
                     wbuf, attv, gsa, gsb, ssn, ssd,
                     sid, 0, ept, 1, OUT, _G2, _STG)

    @pl.when(cid == 1)
    def _():
        _sc_core_run(xl, xr, n0b, d0b, nout1, dout1, att, src_h, dst_h,
                     numer_sh, denom_sh, A3, B3, D3, dsml3, srcstg, dststg,
                     wbuf, attv, gsa, gsb, ssn, ssd,
                     sid, E // 2, ept, 1, OUT, _G2, _STG)


def _make_edge_pass(body, ch, G, stg):
    f32 = jnp.float32
    i32 = jnp.int32
    mesh = plsc.VectorSubcoreMesh(core_axis_name="c", subcore_axis_name="s")
    return pl.kernel(
        body,
        out_type=(
            jax.ShapeDtypeStruct((N, ch), f32),
            jax.ShapeDtypeStruct((N, ch), f32),
            jax.ShapeDtypeStruct((N, _L), f32),
            jax.ShapeDtypeStruct((N, _L), f32),
        ),
        mesh=mesh,
        scratch_types=[
            pltpu.VMEM_SHARED((N, ch), f32),
            pltpu.VMEM_SHARED((N, _L), f32),
            [pltpu.VMEM((G, ch), f32)] * 3,
            [pltpu.VMEM((G, ch), f32)] * 3,
            [pltpu.VMEM((G, _L), f32)] * 3,
            [pltpu.VMEM((G,), i32)] * 3,
            pltpu.VMEM((stg * G,), i32),
            pltpu.VMEM((stg * G,), i32),
            pltpu.VMEM((8, _L), f32),
            pltpu.VMEM((ch,), f32),
            [pltpu.SemaphoreType.DMA] * 3,
            [pltpu.SemaphoreType.DMA] * 3,
            [pltpu.SemaphoreType.DMA] * 3,
            [pltpu.SemaphoreType.DMA] * 3,
        ],
        compiler_params=pltpu.CompilerParams(use_tc_tiling_on_sc=False,
                                             needs_layout_passes=False),
    )


_l1_pass = _make_edge_pass(_l1_body, 128, _G1, _STG)
_l2_pass = _make_edge_pass(_l2_body, OUT, _G2, _STG)


def _decoder_body(z_row, z_col, out_ref):
    acc = lax.dot_general(z_row[...], z_col[...], (((1,), (1,)), ((), ())),
                          preferred_element_type=jnp.float32)
    out_ref[...] = jax.nn.sigmoid(acc)


def _decoder(zp):
    BM = 512
    BN = 512
    grid = (pl.cdiv(N, BM), pl.cdiv(N, BN))
    return pl.pallas_call(
        _decoder_body,
        grid=grid,
        in_specs=[
            pl.BlockSpec((BM, 128), lambda i, j: (i, 0)),
            pl.BlockSpec((BN, 128), lambda i, j: (j, 0)),
        ],
        out_specs=pl.BlockSpec((BM, BN), lambda i, j: (i, j)),
        out_shape=jax.ShapeDtypeStruct((N, N), jnp.float32),
    )(zp, zp)


def _self_loop_init(xl, xr, att, n_heads, chw):
    xlh = xl.reshape(N, n_heads, chw)
    xrh = xr.reshape(N, n_heads, chw)
    t = xlh + xrh
    t = jnp.where(t >= 0.0, t, t * 0.2)
    sw = jnp.exp((t * att.reshape(1, n_heads, chw)).sum(-1))
    numer0 = (sw[:, :, None] * xlh).reshape(N, n_heads * chw)
    lane = jnp.arange(_L)
    denom0 = jnp.where(lane[None, :] < n_heads,
                       sw[:, lane % n_heads], 0.0).astype(jnp.float32)
    return numer0, denom0


def kernel(x, edge_index, W1l, b1l, W1r, b1r, att1, bias1,
           W2l, b2l, W2r, b2r, att2, bias2):
    src = edge_index[0]
    dst = edge_index[1]

    xl = x @ W1l + b1l
    xr = x @ W1r + b1r
    att1f = att1.reshape(H1 * HID)

    n00, d00 = _self_loop_init(xl[:, :128], xr[:, :128], att1f[:128], 4, HID)
    n01, d01 = _self_loop_init(xl[:, 128:], xr[:, 128:], att1f[128:], 4, HID)

    nout0, nout1, dout0, dout1 = _l1_pass(
        xl[:, :128], xl[:, 128:], xr[:, :128], xr[:, 128:],
        n00, n01, d00, d01, src, dst, att1f[:128], att1f[128:])

    numer = jnp.concatenate([nout0, nout1], axis=1).reshape(N, H1, HID)
    denom = jnp.concatenate([dout0[:, :4], dout1[:, :4]], axis=1)
    h1 = (numer / (denom[:, :, None] + 1e-16)).reshape(N, H1 * HID) + bias1

    xl2 = h1 @ W2l + b2l
    xr2 = h1 @ W2r + b2r
    att2f = att2.reshape(OUT)

    n0a, d0a = _self_loop_init(xl2, xr2, att2f, 1, OUT)
    zN16 = jnp.zeros((N, OUT), jnp.float32)
    zN = jnp.zeros((N, _L), jnp.float32)

    n20, n21, d20, d21 = _l2_pass(
        xl2, xr2, n0a, zN16, d0a, zN, src, dst, att2f)

    h2 = (n20 + n21) / (d20[:, :1] + d21[:, :1] + 1e-16) + bias2

    z = h2 / jnp.maximum(jnp.linalg.norm(h2, axis=1, keepdims=True), 1e-12)
    zp = jnp.pad(z, ((0, 0), (0, 128 - OUT)))
    A_pred = _decoder(zp)
    return (A_pred, z)


# speedup vs baseline: 3.0815x; 1.0280x over previous
import functools

import jax
import jax.numpy as jnp
from jax import lax
from jax.experimental import pallas as pl
from jax.experimental.pallas import tpu as pltpu
from jax.experimental.pallas import tpu_sc as plsc

N = 10000
IN = 128
HID = 32
OUT = 16
H1 = 8
E = 320000

_L = 16
_G = 80
_NSUB = 16


def _iota16():
    return lax.broadcasted_iota(jnp.int32, (_L,), 0)


def _group_compute(A, B, D, wbuf, attv, g, n_heads, chw):
    rows = g * _L + _iota16()

    lane = _iota16()
    lanemod = lane % n_heads
    headmask = (lane < n_heads).astype(jnp.float32)
    n_cb = (n_heads * chw) // _L
    zero = jnp.zeros((_L,), jnp.float32)

    ws = []
    for h in range(n_heads):
        accs = [zero, zero, zero, zero]
        for c in range(chw):
            rot = (lane + c) & (chw - 1)
            colv = rot + (h * chw)
            a = plsc.load_gather(A, [rows, colv])
            b = plsc.load_gather(B, [rows, colv])
            av = plsc.load_gather(attv, [colv])
            t = a + b
            t = jnp.maximum(t, t * 0.2)
            accs[c % 4] = accs[c % 4] + av * t
        w = jnp.exp((accs[0] + accs[1]) + (accs[2] + accs[3]))
        wbuf[h, :] = w
        ws.append(w)

    for e in range(_L):
        eab = g * _L + e
        ev = jnp.full((_L,), e, jnp.int32)
        for cb in range(n_cb):
            h = (cb * _L) // chw
            sl = pl.ds(cb * _L, _L)
            A[eab, sl] = A[eab, sl] * ws[h][e]
        wrow = plsc.load_gather(wbuf, [lanemod, ev])
        D[eab, :] = wrow * headmask


def _sc_core_run(xl, xr, n0, d0, nout, dout, att, src_h, dst_h,
                 numer_sh, denom_sh, A3, B3, D3, dsml3, srcstg, dststg,
                 wbuf, attv, gsa, gsb, ssn, ssd,
                 sid, edge_base, ept, n_heads, chw, G, stg):
    nchunks = ept // G
    base0 = edge_base + sid * ept

    pltpu.sync_copy(att, attv)

    @pl.when(sid == 0)
    def _():
        pltpu.sync_copy(n0, numer_sh)
        pltpu.sync_copy(d0, denom_sh)

    def stage(si):
        pltpu.sync_copy(src_h.at[pl.ds(base0 + si * stg * G, stg * G)],
                        srcstg)
        pltpu.sync_copy(dst_h.at[pl.ds(base0 + si * stg * G, stg * G)],
                        dststg)

    def fill_dsml(i, b):
        off = (i % stg) * G
        for j in range(G // _L):
            dsml3[b][pl.ds(j * _L, _L)] = dststg[pl.ds(off + j * _L, _L)]

    def gather_descs(i, b):
        off = (i % stg) * G
        return (
            pltpu.make_async_copy(xl.at[srcstg.at[pl.ds(off, G)]],
                                  A3[b], gsa[b]),
            pltpu.make_async_copy(xr.at[dststg.at[pl.ds(off, G)]],
                                  B3[b], gsb[b]),
        )

    def issue(i, b):
        fill_dsml(i, b)
        da, db = gather_descs(i, b)
        da.start()
        db.start()

    def wait_gathers(i, b):
        da, db = gather_descs(i, b)
        da.wait()
        db.wait()

    def issue_scatter(b):
        pltpu.async_copy(A3[b], numer_sh.at[dsml3[b]], ssn[b], add=True)
        pltpu.async_copy(D3[b], denom_sh.at[dsml3[b]], ssd[b], add=True)

    def wait_scatter(b):
        pltpu.make_async_copy(A3[b], numer_sh.at[dsml3[b]], ssn[b]).wait()
        pltpu.make_async_copy(D3[b], denom_sh.at[dsml3[b]], ssd[b]).wait()

    plsc.subcore_barrier()
    stage(0)
    issue(0, 0)

    def step(i, b):
        nb = (b + 1) % 3
        wait_gathers(i, b)

        @pl.when(i + 1 < nchunks)
        def _():
            @pl.when(i >= 2)
            def _():
                wait_scatter(nb)

            @pl.when((i + 1) % stg == 0)
            def _():
                stage((i + 1) // stg)

            issue(i + 1, nb)

        def group(g, _):
            _group_compute(A3[b], B3[b], D3[b], wbuf, attv, g, n_heads, chw)
            return 0

        lax.fori_loop(0, G // _L, group, 0)
        issue_scatter(b)

    def triple(p, _):
        step(3 * p, 0)
        step(3 * p + 1, 1)
        step(3 * p + 2, 2)
        return 0

    lax.fori_loop(0, nchunks // 3, triple, 0)
    for k in range(nchunks % 3):
        step(3 * (nchunks // 3) + k, k)

    for k in range(3):
        wait_scatter((nchunks - 3 + k) % 3)
    plsc.subcore_barrier()

    @pl.when(sid == 0)
    def _():
        pltpu.sync_copy(numer_sh, nout)
        pltpu.sync_copy(denom_sh, dout)


_G1 = 32
_G2 = 80
_STG = 25


def _l1_body(xl0, xl1, xr0, xr1, n00, n01, d00, d01, src_h, dst_h, att0, att1,
             nout0, nout1, dout0, dout1,
             numer_sh, denom_sh, A3, B3, D3, dsml3, srcstg, dststg,
             wbuf, attv, gsa, gsb, ssn, ssd):
    cid = lax.axis_index("c")
    sid = lax.axis_index("s")
    ept = E // _NSUB

    @pl.when(cid == 0)
    def _():
        _sc_core_run(xl0, xr0, n00, d00, nout0, dout0, att0, src_h, dst_h,
                     numer_sh, denom_sh, A3, B3, D3, dsml3, srcstg, dststg,
                     wbuf, attv, gsa, gsb, ssn, ssd,
                     sid, 0, ept, 4, HID, _G1, _STG)

    @pl.when(cid == 1)
    def _():
        _sc_core_run(xl1, xr1, n01, d01, nout1, dout1, att1, src_h, dst_h,
                     numer_sh, denom_sh, A3, B3, D3, dsml3, srcstg, dststg,
                     wbuf, attv, gsa, gsb, ssn, ssd,
                     sid, 0, ept, 4, HID, _G1, _STG)


def _l2_body(xl, xr, n0a, n0b, d0a, d0b, src_h, dst_h, att,
             nout0, nout1, dout0, dout1,
             numer_sh, denom_sh, A3, B3, D3, dsml3, srcstg, dststg,
             wbuf, attv, gsa, gsb, ssn, ssd):
    cid = lax.axis_index("c")
    sid = lax.axis_index("s")
    ept = (E // 2) // _NSUB

    @pl.when(cid == 0)
    def _():
        _sc_core_run(xl, xr, n0a, d0a, nout0, dout0, att, src_h, dst_h,
                     numer_sh, denom_sh, A3, B3, D3, dsml3, srcstg, dststg,
                     wbuf, attv, gsa, gsb, ssn, ssd,
                     sid, 0, ept, 1, OUT, _G2, _STG)

    @pl.when(cid == 1)
    def _():
        _sc_core_run(xl, xr, n0b, d0b, nout1, dout1, att, src_h, dst_h,
                     numer_sh, denom_sh, A3, B3, D3, dsml3, srcstg, dststg,
                     wbuf, attv, gsa, gsb, ssn, ssd,
                     sid, E // 2, ept, 1, OUT, _G2, _STG)


def _make_edge_pass(body, ch, G, stg):
    f32 = jnp.float32
    i32 = jnp.int32
    mesh = plsc.VectorSubcoreMesh(core_axis_name="c", subcore_axis_name="s")
    return pl.kernel(
        body,
        out_type=(
            jax.ShapeDtypeStruct((N, ch), f32),
            jax.ShapeDtypeStruct((N, ch), f32),
            jax.ShapeDtypeStruct((N, _L), f32),
            jax.ShapeDtypeStruct((N, _L), f32),
        ),
        mesh=mesh,
        scratch_types=[
            pltpu.VMEM_SHARED((N, ch), f32),
            pltpu.VMEM_SHARED((N, _L), f32),
            [pltpu.VMEM((G, ch), f32)] * 3,
            [pltpu.VMEM((G, ch), f32)] * 3,
            [pltpu.VMEM((G, _L), f32)] * 3,
            [pltpu.VMEM((G,), i32)] * 3,
            pltpu.VMEM((stg * G,), i32),
            pltpu.VMEM((stg * G,), i32),
            pltpu.VMEM((8, _L), f32),
            pltpu.VMEM((ch,), f32),
            [pltpu.SemaphoreType.DMA] * 3,
            [pltpu.SemaphoreType.DMA] * 3,
            [pltpu.SemaphoreType.DMA] * 3,
            [pltpu.SemaphoreType.DMA] * 3,
        ],
        compiler_params=pltpu.CompilerParams(use_tc_tiling_on_sc=False,
                                             needs_layout_passes=False),
    )


_l1_pass = _make_edge_pass(_l1_body, 128, _G1, _STG)
_l2_pass = _make_edge_pass(_l2_body, OUT, _G2, _STG)


_BM = 512


def _tc_call(body, n_rowed, out_ch, *arrays):
    in_specs = []
    for k, a in enumerate(arrays):
        if k < n_rowed:
            in_specs.append(pl.BlockSpec((_BM, a.shape[1]),
                                         lambda i: (i, 0)))
        else:
            in_specs.append(pl.BlockSpec(a.shape, lambda i: (0, 0)))
    return pl.pallas_call(
        body,
        grid=(pl.cdiv(N, _BM),),
        in_specs=in_specs,
        out_specs=[pl.BlockSpec((_BM, c), lambda i: (i, 0)) for c in out_ch],
        out_shape=[jax.ShapeDtypeStruct((N, c), jnp.float32)
                   for c in out_ch],
    )(*arrays)


def _prep1_body(x_ref, wl_ref, bl_ref, wr_ref, br_ref, att_ref, m8_ref,
                e8_ref, s0_ref, s1_ref,
                xl0, xl1, xr0, xr1, n00, n01, d00, d01):
    f32 = jnp.float32
    xb = x_ref[...]
    dn = (((1,), (0,)), ((), ()))
    xl = lax.dot_general(xb, wl_ref[...], dn, preferred_element_type=f32)
    xl = xl + bl_ref[...]
    xr = lax.dot_general(xb, wr_ref[...], dn, preferred_element_type=f32)
    xr = xr + br_ref[...]
    t = xl + xr
    t = jnp.maximum(t, t * 0.2)
    ta = t * att_ref[...]
    sl = lax.dot_general(ta, m8_ref[...], dn, preferred_element_type=f32)
    sw = jnp.exp(sl)
    swx = lax.dot_general(sw, e8_ref[...], dn, preferred_element_type=f32)
    n0 = swx * xl
    xl0[...] = xl[:, :128]
    xl1[...] = xl[:, 128:]
    xr0[...] = xr[:, :128]
    xr1[...] = xr[:, 128:]
    n00[...] = n0[:, :128]
    n01[...] = n0[:, 128:]
    d00[...] = lax.dot_general(sw, s0_ref[...], dn, preferred_element_type=f32)
    d01[...] = lax.dot_general(sw, s1_ref[...], dn, preferred_element_type=f32)


def _prep2_body(n0_ref, n1_ref, dp0_ref, dp1_ref, bias1_ref, ed0_ref, ed1_ref,
                wl_ref, bl_ref, wr_ref, br_ref, att_ref, csum_ref, oh_ref,
                xl2, xr2, n0a, d0a):
    f32 = jnp.float32
    dn = (((1,), (0,)), ((), ()))
    den0 = lax.dot_general(dp0_ref[...], ed0_ref[...], dn,
                           preferred_element_type=f32)
    den1 = lax.dot_general(dp1_ref[...], ed1_ref[...], dn,
                           preferred_element_type=f32)
    h1 = jnp.concatenate([n0_ref[...] / (den0 + 1e-16),
                          n1_ref[...] / (den1 + 1e-16)], axis=1)
    h1 = h1 + bias1_ref[...]
    xl = lax.dot_general(h1, wl_ref[...], dn, preferred_element_type=f32)
    xl = xl + bl_ref[...]
    xr = lax.dot_general(h1, wr_ref[...], dn, preferred_element_type=f32)
    xr = xr + br_ref[...]
    t = xl + xr
    t = jnp.maximum(t, t * 0.2)
    sl = lax.dot_general(t * att_ref[...], csum_ref[...], dn,
                         preferred_element_type=f32)
    sw = jnp.exp(sl)
    xl2[...] = xl
    xr2[...] = xr
    n0a[...] = sw * xl
    d0a[...] = sw * oh_ref[...]


def _finish_body(n20_ref, n21_ref, d20_ref, d21_ref, bias2_ref, bc_ref,
                 ones_ref, zp_out):
    f32 = jnp.float32
    dn = (((1,), (0,)), ((), ()))
    den = lax.dot_general(d20_ref[...] + d21_ref[...], bc_ref[...], dn,
                          preferred_element_type=f32)
    h2 = (n20_ref[...] + n21_ref[...]) / (den + 1e-16) + bias2_ref[...]
    nrm = lax.dot_general(h2 * h2, ones_ref[...], dn,
                          preferred_element_type=f32)
    zp_out[...] = h2 / jnp.maximum(jnp.sqrt(nrm), 1e-12)


def _decoder_body(z_row, z_col, out_ref):
    acc = lax.dot_general(z_row[...], z_col[...], (((1,), (1,)), ((), ())),
                          preferred_element_type=jnp.float32)
    out_ref[...] = jax.nn.sigmoid(acc)


def _decoder(zp):
    BM = 512
    BN = 512
    grid = (pl.cdiv(N, BM), pl.cdiv(N, BN))
    return pl.pallas_call(
        _decoder_body,
        grid=grid,
        in_specs=[
            pl.BlockSpec((BM, 128), lambda i, j: (i, 0)),
            pl.BlockSpec((BN, 128), lambda i, j: (j, 0)),
        ],
        out_specs=pl.BlockSpec((BM, BN), lambda i, j: (i, j)),
        out_shape=jax.ShapeDtypeStruct((N, N), jnp.float32),
    )(zp, zp)


def kernel(x, edge_index, W1l, b1l, W1r, b1r, att1, bias1,
           W2l, b2l, W2r, b2r, att2, bias2):
    f32 = jnp.float32
    src = edge_index[0]
    dst = edge_index[1]
    att1f = att1.reshape(H1 * HID)
    att2f = att2.reshape(OUT)

    c256 = jnp.arange(H1 * HID)
    l128 = jnp.arange(128)
    m8 = (c256[:, None] // HID == l128[None, :]).astype(f32)
    e8 = (l128[:, None] == c256[None, :] // HID).astype(f32)
    s0 = ((l128[:, None] == l128[None, :]) & (l128[:, None] < 4)
          ).astype(f32)
    s1 = ((l128[:, None] - 4 == l128[None, :]) & (l128[:, None] >= 4)
          & (l128[:, None] < 8)).astype(f32)
    ed = ((l128[None, :] // HID) == l128[:, None]).astype(f32)
    ed = ed * (l128[:, None] < 4).astype(f32)
    csum = (l128[:, None] < OUT).astype(f32)
    oh = (l128[None, :] == 0).astype(f32).reshape(1, 128)
    bcd = (l128[:, None] == 0).astype(f32)

    xl0, xl1, xr0, xr1, n00, n01, d00p, d01p = _tc_call(
        _prep1_body, 1, [128] * 6 + [128, 128],
        x, W1l, b1l.reshape(1, -1), W1r, b1r.reshape(1, -1),
        att1f.reshape(1, -1), m8, e8, s0, s1)

    nout0, nout1, dout0, dout1 = _l1_pass(
        xl0, xl1, xr0, xr1, n00, n01,
        d00p[:, :_L], d01p[:, :_L], src, dst, att1f[:128], att1f[128:])

    pad112 = ((0, 0), (0, 128 - OUT))
    xl2p, xr2p, n0ap, d0ap = _tc_call(
        _prep2_body, 4, [128] * 4,
        nout0, nout1, jnp.pad(dout0, pad112), jnp.pad(dout1, pad112),
        bias1.reshape(1, -1), ed, ed,
        jnp.pad(W2l, ((0, 0), (0, 128 - OUT))),
        jnp.pad(b2l, (0, 128 - OUT)).reshape(1, -1),
        jnp.pad(W2r, ((0, 0), (0, 128 - OUT))),
        jnp.pad(b2r, (0, 128 - OUT)).reshape(1, -1),
        jnp.pad(att2f, (0, 128 - OUT)).reshape(1, -1), csum, oh)

    zN16 = jnp.zeros((N, OUT), f32)
    zN = jnp.zeros((N, _L), f32)
    n20, n21, d20, d21 = _l2_pass(
        xl2p[:, :OUT], xr2p[:, :OUT], n0ap[:, :OUT], zN16,
        d0ap[:, :_L], zN, src, dst, att2f)

    pad112d = ((0, 0), (0, 128 - _L))
    (zp,) = _tc_call(
        _finish_body, 4, [128],
        jnp.pad(n20, pad112), jnp.pad(n21, pad112),
        jnp.pad(d20, pad112d), jnp.pad(d21, pad112d),
        jnp.pad(bias2, (0, 128 - OUT)).reshape(1, -1), bcd,
        jnp.ones((128, 128), f32))

    A_pred = _decoder(zp)
    return (A_pred, zp[:, :OUT])
